# baseline (speedup 1.0000x reference)
import functools

import jax
import jax.numpy as jnp
from jax import lax
from jax.experimental import pallas as pl
from jax.experimental.pallas import tpu as pltpu
from jax.experimental.pallas import tpu_sc as plsc

B = 16384
L = 50
V = 1000000
D = 128

NC = 2
NS = 16
NW = NC * NS
IPT = B // NW
K = 4
NCH = IPT // K
ROWS = K * L
LANES = 16
STREAMS = ((0, 128), (128, 72))

_mesh = plsc.VectorSubcoreMesh(core_axis_name="c", subcore_axis_name="s")


@functools.partial(
    pl.kernel,
    out_type=jax.ShapeDtypeStruct((B,), jnp.float32),
    mesh=_mesh,
    compiler_params=pltpu.CompilerParams(needs_layout_passes=False),
    scratch_types=[
        pltpu.VMEM((IPT * L,), jnp.int32),
        pltpu.VMEM((ROWS, D), jnp.float32),
        pltpu.VMEM((ROWS, D), jnp.float32),
        pltpu.VMEM((L * D,), jnp.float32),
        pltpu.VMEM((IPT,), jnp.float32),
        pltpu.VMEM((LANES, LANES), jnp.float32),
        pltpu.SemaphoreType.DMA,
        pltpu.SemaphoreType.DMA,
    ],
)
def _glove_sc(x_hbm, w_hbm, table_hbm, out_hbm,
              idx_v, rows0, rows1, w_v, out_v, trans_v, sem0, sem1):
    wid = lax.axis_index("s") * NC + lax.axis_index("c")
    base = wid * IPT
    rows = (rows0, rows1)
    sem = (sem0, sem1)
    pltpu.sync_copy(x_hbm.at[pl.ds(base * L, IPT * L)], idx_v)
    pltpu.sync_copy(w_hbm, w_v)

    def fire(g, b):
        for off, n in STREAMS:
            pltpu.async_copy(
                table_hbm.at[idx_v.at[pl.ds(g * ROWS + off, n)]],
                rows[b].at[pl.ds(off, n)], sem[b],
            )

    def drain(g, b):
        for off, n in STREAMS:
            pltpu.make_async_copy(
                table_hbm.at[idx_v.at[pl.ds(g * ROWS + off, n)]],
                rows[b].at[pl.ds(off, n)], sem[b],
            ).wait()

    def compute(g, b, phase, iout):
        rows_b = rows[b]

        def l_body(l, accs):
            new = list(accs)
            for c in range(D // LANES):
                w = w_v[pl.ds(l * D + c * LANES, LANES)]
                for k in range(K):
                    r = rows_b[k * L + l, pl.ds(c * LANES, LANES)]
                    new[k] = new[k] + r * w
            return tuple(new)

        zero = jnp.zeros((LANES,), jnp.float32)
        accs = lax.fori_loop(0, L, l_body, (zero,) * K)
        for k in range(K):
            trans_v[phase * K + k, :] = accs[k]
        if phase == LANES // K - 1:
            lane = lax.iota(jnp.int32, LANES)
            res = jnp.zeros((LANES,), jnp.float32)
            for c in range(LANES):
                col = jnp.full((LANES,), c, jnp.int32)
                res = res + plsc.load_gather(trans_v, [lane, col])
            out_v[pl.ds(iout, LANES)] = res

    GRP = LANES // K

    fire(0, 0)

    def grp_body(i, carry):
        g0 = i * GRP
        for p in range(GRP):
            g = g0 + p

            @pl.when(g + 1 < NCH)
            def _():
                fire(g + 1, (p + 1) % 2)

            drain(g, p % 2)
            compute(g, p % 2, p, i * LANES)
        return carry

    lax.fori_loop(0, NCH // GRP, grp_body, 0)
    pltpu.sync_copy(out_v, out_hbm.at[pl.ds(base, IPT)])


def kernel(x, table, W, b):
    x_flat = x.reshape(B * L)
    w_flat = W.reshape(L * D)
    out = _glove_sc(x_flat, w_flat, table)
    return out + b[0]
